# baseline (speedup 1.0000x reference)
import functools

import jax
import jax.numpy as jnp
from jax import lax
from jax.experimental import pallas as pl
from jax.experimental.pallas import tpu as pltpu
from jax.experimental.pallas import tpu_sc as plsc


def _make_copy_kernel(sl, d, dtype, nc, ns):
    nw = nc * ns
    rows_per_w = sl // nw
    chunk = rows_per_w
    while chunk * d * 4 * 7 > 448 * 1024:
        chunk //= 2
    n_chunks = rows_per_w // chunk
    nbuf = min(7, n_chunks)
    mesh = plsc.VectorSubcoreMesh(core_axis_name="c", subcore_axis_name="s")

    @functools.partial(
        pl.kernel,
        out_type=jax.ShapeDtypeStruct((sl, d), dtype),
        mesh=mesh,
        scratch_types=[
            pltpu.VMEM((nbuf, chunk, d), dtype),
            pltpu.SemaphoreType.DMA((nbuf,)),
            pltpu.SemaphoreType.DMA((nbuf,)),
        ],
    )
    def copy_rows(table_hbm, out_hbm, buf, in_sem, out_sem):
        wid = lax.axis_index("s") * nc + lax.axis_index("c")
        base = wid * rows_per_w

        def fetch(i):
            b = i % nbuf
            return pltpu.async_copy(
                table_hbm.at[pl.ds(base + i * chunk, chunk)], buf.at[b], in_sem.at[b]
            )

        def flush(i):
            b = i % nbuf
            return pltpu.async_copy(
                buf.at[b], out_hbm.at[pl.ds(base + i * chunk, chunk)], out_sem.at[b]
            )

        fetches = {}
        flushes = {}
        flushed_waited = set()
        prime = max(1, nbuf - 3) if n_chunks > 1 else 1
        for i in range(min(prime, n_chunks)):
            fetches[i] = fetch(i)
        for i in range(n_chunks):
            fetches[i].wait()
            flushes[i] = flush(i)
            nf = i + prime
            if nf < n_chunks:
                prev = nf - nbuf
                if prev >= 0:
                    flushes[prev].wait()
                    flushed_waited.add(prev)
                fetches[nf] = fetch(nf)
        for i in range(n_chunks):
            if i not in flushed_waited:
                flushes[i].wait()

    return copy_rows


def _tc_copy(sl, d, dtype, block):
    def body(in_ref, out_ref):
        out_ref[...] = in_ref[...]

    return pl.pallas_call(
        body,
        grid=(sl // block,),
        in_specs=[pl.BlockSpec((block, d), lambda i: (i, 0))],
        out_specs=pl.BlockSpec((block, d), lambda i: (i, 0)),
        out_shape=jax.ShapeDtypeStruct((sl, d), dtype),
    )


def _tc_dma_copy(sl, d, dtype, nstreams):
    rows = sl // nstreams

    def body(in_ref, out_ref, sems):
        copies = [
            pltpu.make_async_copy(
                in_ref.at[pl.ds(i * rows, rows)],
                out_ref.at[pl.ds(i * rows, rows)],
                sems.at[i],
            )
            for i in range(nstreams)
        ]
        for c in copies:
            c.start()
        for c in copies:
            c.wait()

    return pl.pallas_call(
        body,
        in_specs=[pl.BlockSpec(memory_space=pl.ANY)],
        out_specs=pl.BlockSpec(memory_space=pl.ANY),
        scratch_shapes=[pltpu.SemaphoreType.DMA((nstreams,))],
        out_shape=jax.ShapeDtypeStruct((sl, d), dtype),
    )


def kernel(x, emb_weight):
    sl = x.shape[1]
    _, d = emb_weight.shape
    return _tc_dma_copy(sl, d, emb_weight.dtype, 8)(emb_weight)


# speedup vs baseline: 41.6225x; 41.6225x over previous
import functools

import jax
import jax.numpy as jnp
from jax import lax
from jax.experimental import pallas as pl
from jax.experimental.pallas import tpu as pltpu
from jax.experimental.pallas import tpu_sc as plsc


def _make_copy_kernel(sl, d, dtype, nc, ns):
    nw = nc * ns
    rows_per_w = sl // nw
    chunk = rows_per_w
    while chunk * d * 4 * 7 > 448 * 1024:
        chunk //= 2
    n_chunks = rows_per_w // chunk
    nbuf = min(7, n_chunks)
    mesh = plsc.VectorSubcoreMesh(core_axis_name="c", subcore_axis_name="s")

    @functools.partial(
        pl.kernel,
        out_type=jax.ShapeDtypeStruct((sl, d), dtype),
        mesh=mesh,
        scratch_types=[
            pltpu.VMEM((nbuf, chunk, d), dtype),
            pltpu.SemaphoreType.DMA((nbuf,)),
            pltpu.SemaphoreType.DMA((nbuf,)),
        ],
    )
    def copy_rows(table_hbm, out_hbm, buf, in_sem, out_sem):
        wid = lax.axis_index("s") * nc + lax.axis_index("c")
        base = wid * rows_per_w

        def fetch(i):
            b = i % nbuf
            return pltpu.async_copy(
                table_hbm.at[pl.ds(base + i * chunk, chunk)], buf.at[b], in_sem.at[b]
            )

        def flush(i):
            b = i % nbuf
            return pltpu.async_copy(
                buf.at[b], out_hbm.at[pl.ds(base + i * chunk, chunk)], out_sem.at[b]
            )

        fetches = {}
        flushes = {}
        flushed_waited = set()
        prime = max(1, nbuf - 3) if n_chunks > 1 else 1
        for i in range(min(prime, n_chunks)):
            fetches[i] = fetch(i)
        for i in range(n_chunks):
            fetches[i].wait()
            flushes[i] = flush(i)
            nf = i + prime
            if nf < n_chunks:
                prev = nf - nbuf
                if prev >= 0:
                    flushes[prev].wait()
                    flushed_waited.add(prev)
                fetches[nf] = fetch(nf)
        for i in range(n_chunks):
            if i not in flushed_waited:
                flushes[i].wait()

    return copy_rows


def _tc_copy(sl, d, dtype, block):
    def body(in_ref, out_ref):
        out_ref[...] = in_ref[...]

    return pl.pallas_call(
        body,
        grid=(sl // block,),
        in_specs=[pl.BlockSpec((block, d), lambda i: (i, 0))],
        out_specs=pl.BlockSpec((block, d), lambda i: (i, 0)),
        out_shape=jax.ShapeDtypeStruct((sl, d), dtype),
    )


def _tc_dma_copy(sl, d, dtype, nstreams):
    rows = sl // nstreams

    def body(in_ref, out_ref, sems):
        copies = [
            pltpu.make_async_copy(
                in_ref.at[pl.ds(i * rows, rows)],
                out_ref.at[pl.ds(i * rows, rows)],
                sems.at[i],
            )
            for i in range(nstreams)
        ]
        for c in copies:
            c.start()
        for c in copies:
            c.wait()

    return pl.pallas_call(
        body,
        in_specs=[pl.BlockSpec(memory_space=pl.ANY)],
        out_specs=pl.BlockSpec(memory_space=pl.ANY),
        scratch_shapes=[pltpu.SemaphoreType.DMA((nstreams,))],
        out_shape=jax.ShapeDtypeStruct((sl, d), dtype),
    )


def kernel(x, emb_weight):
    sl = x.shape[1]
    _, d = emb_weight.shape
    return _tc_copy(sl, d, emb_weight.dtype, 1024)(emb_weight)


# speedup vs baseline: 46.4057x; 1.1149x over previous
import functools

import jax
import jax.numpy as jnp
from jax import lax
from jax.experimental import pallas as pl
from jax.experimental.pallas import tpu as pltpu
from jax.experimental.pallas import tpu_sc as plsc


def _make_copy_kernel(sl, d, dtype, nc, ns):
    nw = nc * ns
    rows_per_w = sl // nw
    chunk = rows_per_w
    while chunk * d * 4 * 7 > 448 * 1024:
        chunk //= 2
    n_chunks = rows_per_w // chunk
    nbuf = min(7, n_chunks)
    mesh = plsc.VectorSubcoreMesh(core_axis_name="c", subcore_axis_name="s")

    @functools.partial(
        pl.kernel,
        out_type=jax.ShapeDtypeStruct((sl, d), dtype),
        mesh=mesh,
        scratch_types=[
            pltpu.VMEM((nbuf, chunk, d), dtype),
            pltpu.SemaphoreType.DMA((nbuf,)),
            pltpu.SemaphoreType.DMA((nbuf,)),
        ],
    )
    def copy_rows(table_hbm, out_hbm, buf, in_sem, out_sem):
        wid = lax.axis_index("s") * nc + lax.axis_index("c")
        base = wid * rows_per_w

        def fetch(i):
            b = i % nbuf
            return pltpu.async_copy(
                table_hbm.at[pl.ds(base + i * chunk, chunk)], buf.at[b], in_sem.at[b]
            )

        def flush(i):
            b = i % nbuf
            return pltpu.async_copy(
                buf.at[b], out_hbm.at[pl.ds(base + i * chunk, chunk)], out_sem.at[b]
            )

        fetches = {}
        flushes = {}
        flushed_waited = set()
        prime = max(1, nbuf - 3) if n_chunks > 1 else 1
        for i in range(min(prime, n_chunks)):
            fetches[i] = fetch(i)
        for i in range(n_chunks):
            fetches[i].wait()
            flushes[i] = flush(i)
            nf = i + prime
            if nf < n_chunks:
                prev = nf - nbuf
                if prev >= 0:
                    flushes[prev].wait()
                    flushed_waited.add(prev)
                fetches[nf] = fetch(nf)
        for i in range(n_chunks):
            if i not in flushed_waited:
                flushes[i].wait()

    return copy_rows


def _tc_copy(sl, d, dtype, block):
    def body(in_ref, out_ref):
        out_ref[...] = in_ref[...]

    return pl.pallas_call(
        body,
        grid=(sl // block,),
        in_specs=[pl.BlockSpec((block, d), lambda i: (i, 0))],
        out_specs=pl.BlockSpec((block, d), lambda i: (i, 0)),
        out_shape=jax.ShapeDtypeStruct((sl, d), dtype),
    )


def _tc_dma_copy(sl, d, dtype, nstreams):
    rows = sl // nstreams

    def body(in_ref, out_ref, sems):
        copies = [
            pltpu.make_async_copy(
                in_ref.at[pl.ds(i * rows, rows)],
                out_ref.at[pl.ds(i * rows, rows)],
                sems.at[i],
            )
            for i in range(nstreams)
        ]
        for c in copies:
            c.start()
        for c in copies:
            c.wait()

    return pl.pallas_call(
        body,
        in_specs=[pl.BlockSpec(memory_space=pl.ANY)],
        out_specs=pl.BlockSpec(memory_space=pl.ANY),
        scratch_shapes=[pltpu.SemaphoreType.DMA((nstreams,))],
        out_shape=jax.ShapeDtypeStruct((sl, d), dtype),
    )


def kernel(x, emb_weight):
    sl = x.shape[1]
    _, d = emb_weight.shape
    return _tc_copy(sl, d, emb_weight.dtype, 2048)(emb_weight)
